# baseline (speedup 1.0000x reference)
import jax
import jax.numpy as jnp
from jax.experimental import pallas as pl
from jax.experimental.pallas import tpu as pltpu


def _pick_bm(n: int) -> int:
    for bm in (400, 1000, 200, 100, 40, 8):
        if n % bm == 0:
            return bm
    return n


def _pass1_kernel(x_ref, adj_ref, w1_ref, b1_ref, w2_ref,
                  h_ref, p_ref, s1_scr):
    @pl.when(pl.program_id(0) == 0)
    def _init():
        s1_scr[...] = jnp.dot(x_ref[...], w1_ref[...],
                              preferred_element_type=jnp.float32)

    acc = jnp.dot(adj_ref[...], s1_scr[...],
                  preferred_element_type=jnp.float32)
    h = jnp.maximum(acc + b1_ref[...], 0.0)
    h_ref[...] = h
    p_ref[...] = jnp.dot(h, w2_ref[...], preferred_element_type=jnp.float32)


def _pass2_kernel(adj_ref, p_ref, b2_ref, logz_ref, z_ref):
    z = jnp.dot(adj_ref[...], p_ref[...],
                preferred_element_type=jnp.float32) + b2_ref[...]
    z_ref[...] = z
    m = jnp.max(z, axis=1, keepdims=True)
    logz_ref[...] = (z - m) - jnp.log(
        jnp.sum(jnp.exp(z - m), axis=1, keepdims=True))


@jax.jit
def kernel(x, adj, W1, b1, W2, b2):
    n, nfeat = x.shape
    nhid = W1.shape[1]
    nclass = W2.shape[1]
    bm = _pick_bm(n)
    nblk = n // bm

    row_map = lambda i: (i, 0)
    const_map = lambda i: (0, 0)

    h, p = pl.pallas_call(
        _pass1_kernel,
        grid=(nblk,),
        in_specs=[
            pl.BlockSpec((n, nfeat), const_map),
            pl.BlockSpec((bm, n), row_map),
            pl.BlockSpec((nfeat, nhid), const_map),
            pl.BlockSpec((1, nhid), const_map),
            pl.BlockSpec((nhid, nclass), const_map),
        ],
        out_specs=[
            pl.BlockSpec((bm, nhid), row_map),
            pl.BlockSpec((bm, nclass), row_map),
        ],
        out_shape=[
            jax.ShapeDtypeStruct((n, nhid), jnp.float32),
            jax.ShapeDtypeStruct((n, nclass), jnp.float32),
        ],
        scratch_shapes=[
            pltpu.VMEM((n, nhid), jnp.float32),
        ],
    )(x, adj, W1, b1.reshape(1, nhid), W2)

    logz, z = pl.pallas_call(
        _pass2_kernel,
        grid=(nblk,),
        in_specs=[
            pl.BlockSpec((bm, n), row_map),
            pl.BlockSpec((n, nclass), const_map),
            pl.BlockSpec((1, nclass), const_map),
        ],
        out_specs=[
            pl.BlockSpec((bm, nclass), row_map),
            pl.BlockSpec((bm, nclass), row_map),
        ],
        out_shape=[
            jax.ShapeDtypeStruct((n, nclass), jnp.float32),
            jax.ShapeDtypeStruct((n, nclass), jnp.float32),
        ],
    )(adj, p, b2.reshape(1, nclass))

    return (logz, h, z)
